# baseline (speedup 1.0000x reference)
import jax
import jax.numpy as jnp
from jax.experimental import pallas as pl
from jax.experimental.pallas import tpu as pltpu

MAX_SAMPLES_ = 1000000
N_DIM_ = 64
BATCH_ = 16384

BF = 16
NBF = N_DIM_ // BF
BC = 16384
NBC = -(-MAX_SAMPLES_ // BC)
G2 = (BC - 1 + BATCH_ + BC - 1) // BC
OVC = 2 * G2 * BC


def _body(s_ref, ovl_ref, cache_ref, out_ref):
    jc = pl.program_id(1)
    idx = s_ref[1]
    e1 = s_ref[2]
    e2 = s_ref[3]

    ovl32 = ovl_ref.bitcast(jnp.int32)
    cache32 = cache_ref.bitcast(jnp.int32)
    out32 = out_ref.bitcast(jnp.int32)

    near = ((jc >= s_ref[0]) & (jc < s_ref[0] + G2)) | (jc < G2)

    @pl.when(jnp.logical_not(near))
    def _copy():
        out32[...] = cache32[...]

    @pl.when(near)
    def _merge():
        c = jc * BC + jax.lax.broadcasted_iota(jnp.int32, (BF // 2, BC), 1)
        mask = ((c >= idx) & (c < e1)) | (c < e2)
        out32[...] = jnp.where(mask, ovl32[...], cache32[...])


def kernel(activations, cache, n_valid, index):
    max_samples = cache.shape[0]
    batch = activations.shape[0]
    index = jnp.asarray(index) % max_samples
    new_n_valid = jnp.minimum(jnp.asarray(n_valid) + batch, max_samples)
    new_index = (index + batch) % max_samples

    q1 = (index // BC).astype(jnp.int32)
    im1 = (index % BC).astype(jnp.int32)
    e1 = jnp.minimum(index + batch, max_samples).astype(jnp.int32)
    e2 = (index + batch - max_samples).astype(jnp.int32)
    sp = jnp.stack([q1, index.astype(jnp.int32), e1, e2])

    cache_t = jax.lax.bitcast_convert_type(cache, jnp.bfloat16).T
    acts_t = jax.lax.bitcast_convert_type(
        activations.astype(jnp.float16), jnp.bfloat16).T

    ovl1 = jax.lax.dynamic_update_slice(
        jnp.zeros((N_DIM_, G2 * BC), jnp.bfloat16), acts_t, (0, im1))
    delta = jnp.where(e2 > 0, max_samples - index, batch)
    ovl2 = jax.lax.dynamic_slice(
        jnp.concatenate(
            [acts_t, jnp.zeros((N_DIM_, G2 * BC), jnp.bfloat16)], axis=1),
        (0, delta), (N_DIM_, G2 * BC))
    ovl = jnp.concatenate([ovl1, ovl2], axis=1)

    def ovl_map(f, jc, s):
        in_w1 = (jc >= s[0]) & (jc < s[0] + G2)
        return (f, jnp.where(in_w1, jc - s[0],
                             jnp.where(jc < G2, G2 + jc, 0)))

    grid_spec = pltpu.PrefetchScalarGridSpec(
        num_scalar_prefetch=1,
        grid=(NBF, NBC),
        in_specs=[
            pl.BlockSpec((BF, BC), ovl_map),
            pl.BlockSpec((BF, BC), lambda f, jc, s: (f, jc)),
        ],
        out_specs=pl.BlockSpec((BF, BC), lambda f, jc, s: (f, jc)),
    )

    out_t = pl.pallas_call(
        _body,
        grid_spec=grid_spec,
        out_shape=jax.ShapeDtypeStruct((N_DIM_, max_samples), jnp.bfloat16),
    )(sp, ovl, cache_t)

    new_cache = jax.lax.bitcast_convert_type(out_t.T, jnp.float16)
    return (new_cache, new_n_valid, new_index)


# speedup vs baseline: 1.2090x; 1.2090x over previous
import jax
import jax.numpy as jnp
from jax.experimental import pallas as pl
from jax.experimental.pallas import tpu as pltpu

MAX_SAMPLES_ = 1000000
N_DIM_ = 64
BATCH_ = 16384

BF = 64
NBF = N_DIM_ // BF
BC = 8192
NBC = -(-MAX_SAMPLES_ // BC)
G2 = (BC - 1 + BATCH_ + BC - 1) // BC
OVC = 2 * G2 * BC


def _body(s_ref, ovl_ref, cache_ref, out_ref):
    jc = pl.program_id(1)
    idx = s_ref[1]
    e1 = s_ref[2]
    e2 = s_ref[3]

    ovl32 = ovl_ref.bitcast(jnp.int32)
    cache32 = cache_ref.bitcast(jnp.int32)
    out32 = out_ref.bitcast(jnp.int32)

    near = ((jc >= s_ref[0]) & (jc < s_ref[0] + G2)) | (jc < G2)

    @pl.when(jnp.logical_not(near))
    def _copy():
        out32[...] = cache32[...]

    @pl.when(near)
    def _merge():
        c = jc * BC + jax.lax.broadcasted_iota(jnp.int32, (BF // 2, BC), 1)
        mask = ((c >= idx) & (c < e1)) | (c < e2)
        out32[...] = jnp.where(mask, ovl32[...], cache32[...])


def kernel(activations, cache, n_valid, index):
    max_samples = cache.shape[0]
    batch = activations.shape[0]
    index = jnp.asarray(index) % max_samples
    new_n_valid = jnp.minimum(jnp.asarray(n_valid) + batch, max_samples)
    new_index = (index + batch) % max_samples

    q1 = (index // BC).astype(jnp.int32)
    im1 = (index % BC).astype(jnp.int32)
    e1 = jnp.minimum(index + batch, max_samples).astype(jnp.int32)
    e2 = (index + batch - max_samples).astype(jnp.int32)
    sp = jnp.stack([q1, index.astype(jnp.int32), e1, e2])

    cache_t = jax.lax.bitcast_convert_type(cache, jnp.bfloat16).T
    acts_t = jax.lax.bitcast_convert_type(
        activations.astype(jnp.float16), jnp.bfloat16).T

    ovl1 = jax.lax.dynamic_update_slice(
        jnp.zeros((N_DIM_, G2 * BC), jnp.bfloat16), acts_t, (0, im1))
    delta = jnp.where(e2 > 0, max_samples - index, batch)
    ovl2 = jax.lax.dynamic_slice(
        jnp.concatenate(
            [acts_t, jnp.zeros((N_DIM_, G2 * BC), jnp.bfloat16)], axis=1),
        (0, delta), (N_DIM_, G2 * BC))
    ovl = jnp.concatenate([ovl1, ovl2], axis=1)

    def ovl_map(f, jc, s):
        in_w1 = (jc >= s[0]) & (jc < s[0] + G2)
        return (f, jnp.where(in_w1, jc - s[0],
                             jnp.where(jc < G2, G2 + jc, 0)))

    grid_spec = pltpu.PrefetchScalarGridSpec(
        num_scalar_prefetch=1,
        grid=(NBF, NBC),
        in_specs=[
            pl.BlockSpec((BF, BC), ovl_map),
            pl.BlockSpec((BF, BC), lambda f, jc, s: (f, jc)),
        ],
        out_specs=pl.BlockSpec((BF, BC), lambda f, jc, s: (f, jc)),
    )

    out_t = pl.pallas_call(
        _body,
        grid_spec=grid_spec,
        out_shape=jax.ShapeDtypeStruct((N_DIM_, max_samples), jnp.bfloat16),
    )(sp, ovl, cache_t)

    new_cache = jax.lax.bitcast_convert_type(out_t.T, jnp.float16)
    return (new_cache, new_n_valid, new_index)


# speedup vs baseline: 1.3220x; 1.0934x over previous
import jax
import jax.numpy as jnp
from jax.experimental import pallas as pl
from jax.experimental.pallas import tpu as pltpu

MAX_SAMPLES_ = 1000000
N_DIM_ = 64
BATCH_ = 16384

BF = 64
NBF = N_DIM_ // BF
BC = 16384
NBC = -(-MAX_SAMPLES_ // BC)
G2 = (BC - 1 + BATCH_ + BC - 1) // BC
OVC = 2 * G2 * BC


def _body(s_ref, ovl_ref, cache_ref, out_ref):
    jc = pl.program_id(1)
    idx = s_ref[1]
    e1 = s_ref[2]
    e2 = s_ref[3]

    ovl32 = ovl_ref.bitcast(jnp.int32)
    cache32 = cache_ref.bitcast(jnp.int32)
    out32 = out_ref.bitcast(jnp.int32)

    near = ((jc >= s_ref[0]) & (jc < s_ref[0] + G2)) | (jc < G2)

    @pl.when(jnp.logical_not(near))
    def _copy():
        out32[...] = cache32[...]

    @pl.when(near)
    def _merge():
        c = jc * BC + jax.lax.broadcasted_iota(jnp.int32, (BF // 2, BC), 1)
        mask = ((c >= idx) & (c < e1)) | (c < e2)
        out32[...] = jnp.where(mask, ovl32[...], cache32[...])


def kernel(activations, cache, n_valid, index):
    max_samples = cache.shape[0]
    batch = activations.shape[0]
    index = jnp.asarray(index) % max_samples
    new_n_valid = jnp.minimum(jnp.asarray(n_valid) + batch, max_samples)
    new_index = (index + batch) % max_samples

    q1 = (index // BC).astype(jnp.int32)
    im1 = (index % BC).astype(jnp.int32)
    e1 = jnp.minimum(index + batch, max_samples).astype(jnp.int32)
    e2 = (index + batch - max_samples).astype(jnp.int32)
    sp = jnp.stack([q1, index.astype(jnp.int32), e1, e2])

    cache_t = jax.lax.bitcast_convert_type(cache, jnp.bfloat16).T
    acts_t = jax.lax.bitcast_convert_type(
        activations.astype(jnp.float16), jnp.bfloat16).T

    ovl1 = jax.lax.dynamic_update_slice(
        jnp.zeros((N_DIM_, G2 * BC), jnp.bfloat16), acts_t, (0, im1))
    delta = jnp.where(e2 > 0, max_samples - index, batch)
    ovl2 = jax.lax.dynamic_slice(
        jnp.concatenate(
            [acts_t, jnp.zeros((N_DIM_, G2 * BC), jnp.bfloat16)], axis=1),
        (0, delta), (N_DIM_, G2 * BC))
    ovl = jnp.concatenate([ovl1, ovl2], axis=1)

    def ovl_map(f, jc, s):
        in_w1 = (jc >= s[0]) & (jc < s[0] + G2)
        return (f, jnp.where(in_w1, jc - s[0],
                             jnp.where(jc < G2, G2 + jc, 0)))

    grid_spec = pltpu.PrefetchScalarGridSpec(
        num_scalar_prefetch=1,
        grid=(NBF, NBC),
        in_specs=[
            pl.BlockSpec((BF, BC), ovl_map),
            pl.BlockSpec((BF, BC), lambda f, jc, s: (f, jc)),
        ],
        out_specs=pl.BlockSpec((BF, BC), lambda f, jc, s: (f, jc)),
    )

    out_t = pl.pallas_call(
        _body,
        grid_spec=grid_spec,
        out_shape=jax.ShapeDtypeStruct((N_DIM_, max_samples), jnp.bfloat16),
    )(sp, ovl, cache_t)

    new_cache = jax.lax.bitcast_convert_type(out_t.T, jnp.float16)
    return (new_cache, new_n_valid, new_index)


# speedup vs baseline: 1.3323x; 1.0078x over previous
import jax
import jax.numpy as jnp
from jax.experimental import pallas as pl
from jax.experimental.pallas import tpu as pltpu

MAX_SAMPLES_ = 1000000
N_DIM_ = 64
BATCH_ = 16384

BF = 64
NBF = N_DIM_ // BF
BC = 32768
NBC = -(-MAX_SAMPLES_ // BC)
G2 = (BC - 1 + BATCH_ + BC - 1) // BC
OVC = 2 * G2 * BC


def _body(s_ref, ovl_ref, cache_ref, out_ref):
    jc = pl.program_id(1)
    idx = s_ref[1]
    e1 = s_ref[2]
    e2 = s_ref[3]

    ovl32 = ovl_ref.bitcast(jnp.int32)
    cache32 = cache_ref.bitcast(jnp.int32)
    out32 = out_ref.bitcast(jnp.int32)

    near = ((jc >= s_ref[0]) & (jc < s_ref[0] + G2)) | (jc < G2)

    @pl.when(jnp.logical_not(near))
    def _copy():
        out32[...] = cache32[...]

    @pl.when(near)
    def _merge():
        c = jc * BC + jax.lax.broadcasted_iota(jnp.int32, (BF // 2, BC), 1)
        mask = ((c >= idx) & (c < e1)) | (c < e2)
        out32[...] = jnp.where(mask, ovl32[...], cache32[...])


def kernel(activations, cache, n_valid, index):
    max_samples = cache.shape[0]
    batch = activations.shape[0]
    index = jnp.asarray(index) % max_samples
    new_n_valid = jnp.minimum(jnp.asarray(n_valid) + batch, max_samples)
    new_index = (index + batch) % max_samples

    q1 = (index // BC).astype(jnp.int32)
    im1 = (index % BC).astype(jnp.int32)
    e1 = jnp.minimum(index + batch, max_samples).astype(jnp.int32)
    e2 = (index + batch - max_samples).astype(jnp.int32)
    sp = jnp.stack([q1, index.astype(jnp.int32), e1, e2])

    cache_t = jax.lax.bitcast_convert_type(cache, jnp.bfloat16).T
    acts_t = jax.lax.bitcast_convert_type(
        activations.astype(jnp.float16), jnp.bfloat16).T

    ovl1 = jax.lax.dynamic_update_slice(
        jnp.zeros((N_DIM_, G2 * BC), jnp.bfloat16), acts_t, (0, im1))
    delta = jnp.where(e2 > 0, max_samples - index, batch)
    ovl2 = jax.lax.dynamic_slice(
        jnp.concatenate(
            [acts_t, jnp.zeros((N_DIM_, G2 * BC), jnp.bfloat16)], axis=1),
        (0, delta), (N_DIM_, G2 * BC))
    ovl = jnp.concatenate([ovl1, ovl2], axis=1)

    def ovl_map(f, jc, s):
        in_w1 = (jc >= s[0]) & (jc < s[0] + G2)
        return (f, jnp.where(in_w1, jc - s[0],
                             jnp.where(jc < G2, G2 + jc, 0)))

    grid_spec = pltpu.PrefetchScalarGridSpec(
        num_scalar_prefetch=1,
        grid=(NBF, NBC),
        in_specs=[
            pl.BlockSpec((BF, BC), ovl_map),
            pl.BlockSpec((BF, BC), lambda f, jc, s: (f, jc)),
        ],
        out_specs=pl.BlockSpec((BF, BC), lambda f, jc, s: (f, jc)),
    )

    out_t = pl.pallas_call(
        _body,
        grid_spec=grid_spec,
        out_shape=jax.ShapeDtypeStruct((N_DIM_, max_samples), jnp.bfloat16),
    )(sp, ovl, cache_t)

    new_cache = jax.lax.bitcast_convert_type(out_t.T, jnp.float16)
    return (new_cache, new_n_valid, new_index)
